# baseline (device time: 19087 ns/iter reference)
import jax
import jax.numpy as jnp
from jax import lax
from jax.experimental import pallas as pl
from jax.experimental.pallas import tpu as pltpu

K = 4


def kernel(x):
    m, n = x.shape
    nh = n // 2
    r = m // K

    def body(x_hbm, out_hbm, xv, sendb, locb,
             in_sems, send_sems, recv_sems, loc_sems):
        my_x = lax.axis_index("x")
        my_y = lax.axis_index("y")
        my_z = lax.axis_index("z")

        in_copies = [
            pltpu.make_async_copy(
                x_hbm.at[i * r:(i + 1) * r, :],
                xv.at[i * r:(i + 1) * r, :],
                in_sems.at[i],
            )
            for i in range(K)
        ]
        for c in in_copies:
            c.start()

        barrier_sem = pltpu.get_barrier_semaphore()

        def exchange(y):
            peer = (my_x, 1 - y, my_z)
            pl.semaphore_signal(
                barrier_sem, inc=1,
                device_id=peer, device_id_type=pl.DeviceIdType.MESH,
            )
            pl.semaphore_wait(barrier_sem, 1)

            rdmas = []
            for i in range(K):
                rows = slice(i * r, (i + 1) * r)
                in_copies[i].wait()
                sendb[rows, :] = xv[rows, (1 - y) * nh:(2 - y) * nh].astype(
                    jnp.bfloat16)
                rdma = pltpu.make_async_remote_copy(
                    src_ref=sendb.at[rows, :],
                    dst_ref=out_hbm.at[y * m + i * r:y * m + (i + 1) * r, :],
                    send_sem=send_sems.at[i],
                    recv_sem=recv_sems.at[i],
                    device_id=peer,
                    device_id_type=pl.DeviceIdType.MESH,
                )
                rdma.start()
                rdmas.append(rdma)

            loc_copies = []
            for i in range(K):
                rows = slice(i * r, (i + 1) * r)
                locb[rows, :] = xv[rows, y * nh:(y + 1) * nh].astype(
                    jnp.bfloat16)
                c = pltpu.make_async_copy(
                    locb.at[rows, :],
                    out_hbm.at[y * m + i * r:y * m + (i + 1) * r, :],
                    loc_sems.at[i],
                )
                c.start()
                loc_copies.append(c)

            for c in loc_copies:
                c.wait()
            for rdma in rdmas:
                rdma.wait_send()
            for rdma in rdmas:
                rdma.wait_recv()

        @pl.when(my_y == 0)
        def _():
            exchange(0)

        @pl.when(my_y == 1)
        def _():
            exchange(1)

    return pl.pallas_call(
        body,
        out_shape=jax.ShapeDtypeStruct((2 * m, nh), jnp.bfloat16),
        in_specs=[pl.BlockSpec(memory_space=pl.ANY)],
        out_specs=pl.BlockSpec(memory_space=pl.ANY),
        scratch_shapes=[
            pltpu.VMEM((m, n), jnp.float32),
            pltpu.VMEM((m, nh), jnp.bfloat16),
            pltpu.VMEM((m, nh), jnp.bfloat16),
            pltpu.SemaphoreType.DMA((K,)),
            pltpu.SemaphoreType.DMA((K,)),
            pltpu.SemaphoreType.DMA((K,)),
            pltpu.SemaphoreType.DMA((K,)),
        ],
        compiler_params=pltpu.CompilerParams(collective_id=0),
    )(x)
